# baseline (speedup 1.0000x reference)
import functools

import jax
import jax.numpy as jnp
from jax import lax
from jax.experimental import pallas as pl
from jax.experimental.pallas import tpu as pltpu
from jax.experimental.pallas import tpu_sc as plsc

N_ROWS = 1000000
B = 16384
NC, NS, L = 2, 16, 16
NW = NC * NS
BPW = B // NW
CHUNK = 128
NCHUNK = BPW // CHUNK
GROUPS = BPW // L


def _rsqrt(s):
    i = plsc.bitcast(s, jnp.int32)
    y = plsc.bitcast(jnp.int32(0x5F3759DF) - (i >> 1), jnp.float32)
    for _ in range(3):
        y = y * (1.5 - (0.5 * s) * y * y)
    return y


def _body(rots_hbm, trans_hbm, ind_hbm, rot_hbm, tran_hbm,
          idx_v, exp_v, rows_v, tr_v, rot_v, sem):
    wid = lax.axis_index("s") * NC + lax.axis_index("c")
    base = wid * BPW
    for j in range(NCHUNK):
        pltpu.sync_copy(ind_hbm.at[pl.ds(base + j * CHUNK, CHUNK)],
                        idx_v.at[j])

    iota = jnp.arange(L, dtype=jnp.int32)

    zero = jnp.zeros((L,), jnp.int32)
    for k in range(4):
        for q in range(4 * BPW // L):
            exp_v[k, pl.ds(q * L, L)] = zero
    for j in range(NCHUNK):
        for g in range(CHUNK // L):
            v = idx_v[j, pl.ds(g * L, L)]
            pos = 4 * ((j * CHUNK + g * L) + iota)
            plsc.store_scatter(exp_v.at[0], [pos], v * 12)
            plsc.store_scatter(exp_v.at[1], [pos], v * 12 + 4)
            plsc.store_scatter(exp_v.at[2], [pos], v * 12 + 8)
            plsc.store_scatter(exp_v.at[3], [pos], v * 4)

    copies = [pltpu.async_copy(rots_hbm.at[exp_v.at[k]],
                               rows_v.at[k], sem) for k in range(3)]
    copies.append(pltpu.async_copy(trans_hbm.at[exp_v.at[3]], tr_v, sem))
    for c in copies:
        c.wait()
    cols2 = [jnp.full((L,), c, jnp.int32) for c in range(2)]
    cols9 = [jnp.full((L,), c, jnp.int32) for c in range(9)]

    def group(g, carry):
        rows = g * L + iota
        a = [plsc.load_gather(rows_v.at[k], [rows, cols2[c]])
             for k in range(3) for c in range(2)]
        s1 = a[0] * a[0] + a[1] * a[1] + a[2] * a[2]
        n1 = jnp.maximum(s1 * _rsqrt(s1), 1e-5)
        inv1 = 1.0 / n1
        e1 = [a[0] * inv1, a[1] * inv1, a[2] * inv1]
        d = e1[0] * a[3] + e1[1] * a[4] + e1[2] * a[5]
        u = [a[3] - d * e1[0], a[4] - d * e1[1], a[5] - d * e1[2]]
        s2 = u[0] * u[0] + u[1] * u[1] + u[2] * u[2]
        n2 = jnp.maximum(s2 * _rsqrt(s2), 1e-5)
        inv2 = 1.0 / n2
        e2 = [u[0] * inv2, u[1] * inv2, u[2] * inv2]
        e3 = [e1[1] * e2[2] - e1[2] * e2[1],
              e1[2] * e2[0] - e1[0] * e2[2],
              e1[0] * e2[1] - e1[1] * e2[0]]
        out = e1 + e2 + e3
        for c in range(9):
            plsc.store_scatter(rot_v, [rows, cols9[c]], out[c])
        return carry

    lax.fori_loop(0, GROUPS, group, 0)

    pltpu.sync_copy(rot_v, rot_hbm.at[pl.ds(base, BPW)])
    pltpu.sync_copy(tr_v.at[pl.ds(0, BPW)], tran_hbm.at[pl.ds(base, BPW)])


@functools.partial(
    pl.kernel,
    out_type=(jax.ShapeDtypeStruct((B, 9), jnp.float32),
              jax.ShapeDtypeStruct((B, 2), jnp.float32)),
    mesh=plsc.VectorSubcoreMesh(core_axis_name="c", subcore_axis_name="s"),
    compiler_params=pltpu.CompilerParams(needs_layout_passes=False,
                                         use_tc_tiling_on_sc=False),
    scratch_types=[
        pltpu.VMEM((NCHUNK, CHUNK), jnp.int32),
        pltpu.VMEM((4, 4 * BPW), jnp.int32),
        pltpu.VMEM((3, 4 * BPW, 2), jnp.float32),
        pltpu.VMEM((4 * BPW, 2), jnp.float32),
        pltpu.VMEM((BPW, 9), jnp.float32),
        pltpu.SemaphoreType.DMA,
    ],
)
def _pose_kernel(rots_hbm, trans_hbm, ind_hbm, rot_hbm, tran_hbm,
                 idx_v, exp_v, rows_v, tr_v, rot_v, sem):
    _body(rots_hbm, trans_hbm, ind_hbm, rot_hbm, tran_hbm,
          idx_v, exp_v, rows_v, tr_v, rot_v, sem)


def kernel(rots_emb_weight, trans_emb_weight, ind):
    rot9, tran = _pose_kernel(rots_emb_weight.reshape(3 * N_ROWS, 2),
                              trans_emb_weight, ind.astype(jnp.int32))
    return rot9.reshape(B, 3, 3), tran


# speedup vs baseline: 3.0354x; 3.0354x over previous
import functools

import jax
import jax.numpy as jnp
from jax import lax
from jax.experimental import pallas as pl
from jax.experimental.pallas import tpu as pltpu
from jax.experimental.pallas import tpu_sc as plsc

N_ROWS = 1000000
B = 16384
NC, NS, L = 2, 16, 16
NW = NC * NS
BPW = B // NW
GROUPS = BPW // L


def _rsqrt(s):
    i = plsc.bitcast(s, jnp.int32)
    y = plsc.bitcast(jnp.int32(0x5F3759DF) - (i >> 1), jnp.float32)
    for _ in range(3):
        y = y * (1.5 - (0.5 * s) * y * y)
    return y


def _body(rots_hbm, trans_hbm, ind_hbm, rot_hbm, tran_hbm,
          idx_v, exp_v, comp_v, rot_v, tr_v, sem):
    wid = lax.axis_index("s") * NC + lax.axis_index("c")
    base = wid * BPW
    pltpu.sync_copy(ind_hbm.at[pl.ds(base, BPW)], idx_v)

    iota = jnp.arange(L, dtype=jnp.int32)
    for g in range(GROUPS):
        v = idx_v[pl.ds(g * L, L)]
        pos = 4 * (g * L + iota)
        v24 = v * 24
        v8 = v * 8
        for c in range(6):
            plsc.store_scatter(exp_v.at[c], [pos], v24 + 4 * c)
        for k in range(2):
            plsc.store_scatter(exp_v.at[6 + k], [pos], v8 + 4 * k)

    copies = [pltpu.async_copy(rots_hbm.at[exp_v.at[c]],
                               comp_v.at[c], sem) for c in range(6)]
    copies += [pltpu.async_copy(trans_hbm.at[exp_v.at[6 + k]],
                                comp_v.at[6 + k], sem) for k in range(2)]
    for c in copies:
        c.wait()

    for g in range(GROUPS):
        sl = pl.ds(g * L, L)
        a = [comp_v[c, sl] for c in range(6)]
        t = [comp_v[6 + k, sl] for k in range(2)]
        s1 = a[0] * a[0] + a[1] * a[1] + a[2] * a[2]
        n1 = jnp.maximum(s1 * _rsqrt(s1), 1e-5)
        inv1 = 1.0 / n1
        e1 = [a[0] * inv1, a[1] * inv1, a[2] * inv1]
        d = e1[0] * a[3] + e1[1] * a[4] + e1[2] * a[5]
        u = [a[3] - d * e1[0], a[4] - d * e1[1], a[5] - d * e1[2]]
        s2 = u[0] * u[0] + u[1] * u[1] + u[2] * u[2]
        n2 = jnp.maximum(s2 * _rsqrt(s2), 1e-5)
        inv2 = 1.0 / n2
        e2 = [u[0] * inv2, u[1] * inv2, u[2] * inv2]
        e3 = [e1[1] * e2[2] - e1[2] * e2[1],
              e1[2] * e2[0] - e1[0] * e2[2],
              e1[0] * e2[1] - e1[1] * e2[0]]
        out = e1 + e2 + e3
        pos9 = 9 * (g * L + iota)
        for c in range(9):
            plsc.store_scatter(rot_v, [pos9 + c], out[c])
        pos2 = 2 * (g * L + iota)
        for k in range(2):
            plsc.store_scatter(tr_v, [pos2 + k], t[k])

    pltpu.sync_copy(rot_v, rot_hbm.at[pl.ds(base * 9, BPW * 9)])
    pltpu.sync_copy(tr_v, tran_hbm.at[pl.ds(base * 2, BPW * 2)])


@functools.partial(
    pl.kernel,
    out_type=(jax.ShapeDtypeStruct((B * 9,), jnp.float32),
              jax.ShapeDtypeStruct((B * 2,), jnp.float32)),
    mesh=plsc.VectorSubcoreMesh(core_axis_name="c", subcore_axis_name="s"),
    compiler_params=pltpu.CompilerParams(needs_layout_passes=False,
                                         use_tc_tiling_on_sc=False),
    scratch_types=[
        pltpu.VMEM((BPW,), jnp.int32),
        pltpu.VMEM((8, 4 * BPW), jnp.int32),
        pltpu.VMEM((8, 4 * BPW), jnp.float32),
        pltpu.VMEM((BPW * 9,), jnp.float32),
        pltpu.VMEM((BPW * 2,), jnp.float32),
        pltpu.SemaphoreType.DMA,
    ],
)
def _pose_kernel(rots_hbm, trans_hbm, ind_hbm, rot_hbm, tran_hbm,
                 idx_v, exp_v, comp_v, rot_v, tr_v, sem):
    _body(rots_hbm, trans_hbm, ind_hbm, rot_hbm, tran_hbm,
          idx_v, exp_v, comp_v, rot_v, tr_v, sem)


def kernel(rots_emb_weight, trans_emb_weight, ind):
    rot9, tran = _pose_kernel(rots_emb_weight.reshape(6 * N_ROWS),
                              trans_emb_weight.reshape(2 * N_ROWS),
                              ind.astype(jnp.int32))
    return rot9.reshape(B, 3, 3), tran.reshape(B, 2)
